# Initial kernel scaffold; baseline (speedup 1.0000x reference)
import jax
import jax.numpy as jnp
from jax.experimental import pallas as pl


def kernel(x, edge_index, edge_attr, batch, enc_W, enc_b, edge_W, edge_b, msg_W1, msg_b1, msg_W2, msg_b2, gru_Wi, gru_bi, gru_Wh, gru_bh, bn_g, bn_b, skip_W, skip_b, r_W1, r_b1, r_W2, r_b2, r_W3, r_b3):
    raise NotImplementedError("write your pallas kernel here")


# speedup vs baseline: 3.7965x; 3.7965x over previous
import functools

import jax
import jax.numpy as jnp
from jax import lax
from jax.experimental import pallas as pl
from jax.experimental.pallas import tpu as pltpu
from jax.experimental.pallas import tpu_sc as plsc

_NC = 2
_NS = 16
_LN = 16
_CHUNK = 80
_G = 64

_f32 = jnp.float32


def _sig(v):
    return 1.0 / (1.0 + jnp.exp(-v))


def _dot(a, b):
    return jnp.dot(a, b, preferred_element_type=_f32)


def _tc_encode(x, enc_W, enc_b, wa, wb):
    n, h = x.shape[0], enc_W.shape[1]

    def body(x_ref, w_ref, b_ref, wa_ref, wb_ref, h_ref, a_ref, b2_ref):
        hv = jnp.maximum(_dot(x_ref[...], w_ref[...]) + b_ref[...], 0.0)
        h_ref[...] = hv
        a_ref[...] = _dot(hv, wa_ref[...])
        b2_ref[...] = _dot(hv, wb_ref[...])

    sds = jax.ShapeDtypeStruct((n, h), _f32)
    return pl.pallas_call(body, out_shape=(sds, sds, sds))(x, enc_W, enc_b, wa, wb)


def _tc_edgefeat(edge_attr, edge_W, edge_b, wc, b1):
    e, bf = edge_attr.shape
    h = wc.shape[1]
    blk = 2000

    def body(ea_ref, w_ref, b_ref, wc_ref, b1_ref, o_ref):
        ev = jnp.maximum(_dot(ea_ref[...], w_ref[...]) + b_ref[...], 0.0)
        o_ref[...] = _dot(ev, wc_ref[...]) + b1_ref[...]

    return pl.pallas_call(
        body,
        grid=(e // blk,),
        in_specs=[
            pl.BlockSpec((blk, bf), lambda i: (i, 0)),
            pl.BlockSpec((bf, h), lambda i: (0, 0)),
            pl.BlockSpec((1, h), lambda i: (0, 0)),
            pl.BlockSpec((h, h), lambda i: (0, 0)),
            pl.BlockSpec((1, h), lambda i: (0, 0)),
        ],
        out_specs=pl.BlockSpec((blk, h), lambda i: (i, 0)),
        out_shape=jax.ShapeDtypeStruct((e, h), _f32),
    )(edge_attr, edge_W, edge_b, wc, b1)


def _tc_layer(s2, c2, h, w2, b2, wi, bi, wh, bh, bng, bnb, skw, skb, wa=None, wb=None):
    n, hd = h.shape
    emit_ab = wa is not None

    def body(*refs):
        if emit_ab:
            (s2_ref, c2_ref, h_ref, w2_ref, b2_ref, wi_ref, bi_ref, wh_ref,
             bh_ref, bng_ref, bnb_ref, skw_ref, skb_ref, wa_ref, wb_ref,
             ho_ref, ao_ref, bo_ref) = refs
        else:
            (s2_ref, c2_ref, h_ref, w2_ref, b2_ref, wi_ref, bi_ref, wh_ref,
             bh_ref, bng_ref, bnb_ref, skw_ref, skb_ref, ho_ref) = refs
        s2v = s2_ref[...]
        sv = s2v[0] + s2v[1]
        c2v = c2_ref[...]
        counts = c2v[0, :, 0:1] + c2v[1, :, 0:1]
        hv = h_ref[...]
        aggr = _dot(sv, w2_ref[...]) + counts * b2_ref[...]
        wiv, whv = wi_ref[...], wh_ref[...]
        biv, bhv = bi_ref[...], bh_ref[...]
        r = _sig(_dot(aggr, wiv[:, :hd]) + biv[:, :hd]
                 + _dot(hv, whv[:, :hd]) + bhv[:, :hd])
        z = _sig(_dot(aggr, wiv[:, hd:2 * hd]) + biv[:, hd:2 * hd]
                 + _dot(hv, whv[:, hd:2 * hd]) + bhv[:, hd:2 * hd])
        nn = jnp.tanh(_dot(aggr, wiv[:, 2 * hd:]) + biv[:, 2 * hd:]
                      + r * (_dot(hv, whv[:, 2 * hd:]) + bhv[:, 2 * hd:]))
        hn = (1.0 - z) * nn + z * hv
        mu = jnp.mean(hn, axis=0, keepdims=True)
        var = jnp.mean((hn - mu) ** 2, axis=0, keepdims=True)
        hb = (hn - mu) / jnp.sqrt(var + 1e-5) * bng_ref[...] + bnb_ref[...]
        ho = jnp.maximum(hb, 0.0) + _dot(hv, skw_ref[...]) + skb_ref[...]
        ho_ref[...] = ho
        if emit_ab:
            ao_ref[...] = _dot(ho, wa_ref[...])
            bo_ref[...] = _dot(ho, wb_ref[...])

    sds = jax.ShapeDtypeStruct((n, hd), _f32)
    args = [s2, c2, h, w2, b2, wi, bi, wh, bh, bng, bnb, skw, skb]
    if emit_ab:
        args += [wa, wb]
        return pl.pallas_call(body, out_shape=(sds, sds, sds))(*args)
    return pl.pallas_call(body, out_shape=sds)(*args)


def _tc_pool(h, batch2d, w1, b1, w2, b2, w3, b3):
    n, hd = h.shape

    def body(h_ref, bt_ref, w1_ref, b1_ref, w2_ref, b2_ref, w3_ref, b3_ref, o_ref):
        hv = h_ref[...]
        bt = bt_ref[...]
        gids = lax.broadcasted_iota(jnp.int32, (n, _G), 1)
        onehot = (bt == gids).astype(_f32)
        ssum = lax.dot_general(onehot, hv, (((0,), (0,)), ((), ())),
                               preferred_element_type=_f32)
        cnt = jnp.sum(onehot, axis=0)[:, None]
        hmean = ssum / jnp.maximum(cnt, 1.0)
        rows = []
        for g in range(_G):
            rows.append(jnp.max(jnp.where(bt == g, hv, -3.4e38), axis=0,
                                keepdims=True))
        hmax = jnp.concatenate(rows, axis=0)
        hg = jnp.concatenate([hmean, hmax], axis=1)
        o = jnp.maximum(_dot(hg, w1_ref[...]) + b1_ref[...], 0.0)
        o = jnp.maximum(_dot(o, w2_ref[...]) + b2_ref[...], 0.0)
        o_ref[...] = _sig(_dot(o, w3_ref[...]) + b3_ref[...])

    return pl.pallas_call(
        body, out_shape=jax.ShapeDtypeStruct((_G, 1), _f32),
    )(h, batch2d, w1, b1, w2, b2, w3, b3)


def _sc_counts(dst3d, zc, ones_c):
    nw, ch, ck = dst3d.shape
    n = zc.shape[0] * _NS
    rt = (n // _NS) // 8 * 8
    rem = n - rt * _NS
    mesh = plsc.VectorSubcoreMesh(core_axis_name="c", subcore_axis_name="s")

    @functools.partial(
        pl.kernel, mesh=mesh,
        out_type=jax.ShapeDtypeStruct((_NC, n, _LN), _f32),
        scratch_types=[
            pltpu.VMEM((ch, ck), jnp.int32),
            pltpu.VMEM((ck, _LN), _f32),
            pltpu.VMEM_SHARED((n, _LN), _f32),
        ])
    def k(dst_hbm, zc_hbm, on_hbm, c_out, idxd, onesv, c_sh):
        ci = lax.axis_index("c")
        si = lax.axis_index("s")
        wid = ci * _NS + si
        pltpu.sync_copy(dst_hbm.at[wid], idxd)
        pltpu.sync_copy(on_hbm, onesv)
        pltpu.sync_copy(zc_hbm.at[pl.ds(0, rt)], c_sh.at[pl.ds(si * rt, rt)])

        @pl.when(si == _NS - 1)
        def _():
            pltpu.sync_copy(zc_hbm.at[pl.ds(0, rem)],
                            c_sh.at[pl.ds(_NS * rt, rem)])

        plsc.subcore_barrier()

        @pl.loop(0, ch)
        def _(i):
            pltpu.sync_copy(onesv, c_sh.at[idxd.at[i]], add=True)

        plsc.subcore_barrier()
        pltpu.sync_copy(c_sh.at[pl.ds(si * rt, rt)],
                        c_out.at[ci, pl.ds(si * rt, rt)])

        @pl.when(si == _NS - 1)
        def _():
            pltpu.sync_copy(c_sh.at[pl.ds(_NS * rt, rem)],
                            c_out.at[ci, pl.ds(_NS * rt, rem)])

    return k(dst3d, zc, ones_c)


def _sc_edge_layer(a, b, ec, dst3d, src3d, zs):
    n, hd = a.shape
    nw, ch, ck = dst3d.shape
    ew = ch * ck
    rt = (n // _NS) // 8 * 8
    rem = n - rt * _NS
    mesh = plsc.VectorSubcoreMesh(core_axis_name="c", subcore_axis_name="s")

    @functools.partial(
        pl.kernel, mesh=mesh,
        out_type=jax.ShapeDtypeStruct((_NC, n, hd), _f32),
        scratch_types=[
            pltpu.VMEM((2, ck), jnp.int32),
            pltpu.VMEM((2, ck), jnp.int32),
            pltpu.VMEM((ck, hd), _f32),
            pltpu.VMEM((ck, hd), _f32),
            pltpu.VMEM((ck, hd), _f32),
            pltpu.VMEM_SHARED((n, hd), _f32),
            pltpu.SemaphoreType.DMA,
            pltpu.SemaphoreType.DMA,
            pltpu.SemaphoreType.DMA,
            pltpu.SemaphoreType.DMA,
            pltpu.SemaphoreType.DMA,
        ])
    def k(a_hbm, b_hbm, ec_hbm, dst_hbm, src_hbm, zs_hbm,
          s_out, idxd2, idxs2, av, bv, ecv, s_sh,
          sem_a, sem_b, sem_e, sem_di, sem_si):
        ci = lax.axis_index("c")
        si = lax.axis_index("s")
        wid = ci * _NS + si
        pltpu.sync_copy(zs_hbm.at[pl.ds(0, rt)], s_sh.at[pl.ds(si * rt, rt)])

        @pl.when(si == _NS - 1)
        def _():
            pltpu.sync_copy(zs_hbm.at[pl.ds(0, rem)],
                            s_sh.at[pl.ds(_NS * rt, rem)])

        pltpu.async_copy(dst_hbm.at[wid, 0], idxd2.at[0], sem_di)
        pltpu.async_copy(src_hbm.at[wid, 0], idxs2.at[0], sem_si)
        plsc.subcore_barrier()

        @pl.loop(0, ch)
        def _(i):
            cur = lax.rem(i, 2)
            pltpu.make_async_copy(dst_hbm.at[wid, i], idxd2.at[cur],
                                  sem_di).wait()
            pltpu.make_async_copy(src_hbm.at[wid, i], idxs2.at[cur],
                                  sem_si).wait()

            @pl.when(i + 1 < ch)
            def _():
                pltpu.async_copy(dst_hbm.at[wid, i + 1], idxd2.at[1 - cur],
                                 sem_di)
                pltpu.async_copy(src_hbm.at[wid, i + 1], idxs2.at[1 - cur],
                                 sem_si)

            di = idxd2.at[cur]
            sri = idxs2.at[cur]
            cp_a = pltpu.async_copy(a_hbm.at[di], av, sem_a)
            cp_b = pltpu.async_copy(b_hbm.at[sri], bv, sem_b)
            cp_e = pltpu.async_copy(ec_hbm.at[pl.ds(wid * ew + i * ck, ck)],
                                    ecv, sem_e)
            cp_a.wait()
            cp_b.wait()
            cp_e.wait()

            @pl.loop(0, ck)
            def _(r):
                for j in range(hd // _LN):
                    sl = pl.ds(j * _LN, _LN)
                    av[r, sl] = jnp.maximum(av[r, sl] + bv[r, sl] + ecv[r, sl],
                                            0.0)

            pltpu.sync_copy(av, s_sh.at[di], add=True)

        plsc.subcore_barrier()
        pltpu.sync_copy(s_sh.at[pl.ds(si * rt, rt)],
                        s_out.at[ci, pl.ds(si * rt, rt)])

        @pl.when(si == _NS - 1)
        def _():
            pltpu.sync_copy(s_sh.at[pl.ds(_NS * rt, rem)],
                            s_out.at[ci, pl.ds(_NS * rt, rem)])

    return k(a, b, ec, dst3d, src3d, zs)


def kernel(x, edge_index, edge_attr, batch, enc_W, enc_b, edge_W, edge_b,
           msg_W1, msg_b1, msg_W2, msg_b2, gru_Wi, gru_bi, gru_Wh, gru_bh,
           bn_g, bn_b, skip_W, skip_b, r_W1, r_b1, r_W2, r_b2, r_W3, r_b3):
    n, hd = x.shape[0], enc_W.shape[1]
    e = edge_attr.shape[0]
    nl = msg_W1.shape[0]
    nw = _NC * _NS
    ew = e // nw
    ch = ew // _CHUNK

    src = edge_index[0]
    dst = edge_index[1]
    dst2d = dst.reshape(nw, ch, _CHUNK)
    src2d = src.reshape(nw, ch, _CHUNK)

    w1a = [msg_W1[l, :hd] for l in range(nl)]
    w1b = [msg_W1[l, hd:2 * hd] for l in range(nl)]
    w1c = [msg_W1[l, 2 * hd:] for l in range(nl)]
    b1 = [msg_b1[l].reshape(1, hd) for l in range(nl)]

    zs = jnp.zeros((n // _NS, hd), _f32)
    zc = jnp.zeros((n // _NS, _LN), _f32)
    ones_c = jnp.ones((_CHUNK, _LN), _f32)

    h, a, b = _tc_encode(x, enc_W, enc_b.reshape(1, hd), w1a[0], w1b[0])
    ec = [_tc_edgefeat(edge_attr, edge_W, edge_b.reshape(1, hd), w1c[l], b1[l])
          for l in range(nl)]
    c2 = _sc_counts(dst2d, zc, ones_c)

    for l in range(nl):
        s2 = _sc_edge_layer(a, b, ec[l], dst2d, src2d, zs)
        common = (s2, c2, h, msg_W2[l], msg_b2[l].reshape(1, hd),
                  gru_Wi[l], gru_bi[l].reshape(1, 3 * hd),
                  gru_Wh[l], gru_bh[l].reshape(1, 3 * hd),
                  bn_g[l].reshape(1, hd), bn_b[l].reshape(1, hd),
                  skip_W[l], skip_b[l].reshape(1, hd))
        if l < nl - 1:
            h, a, b = _tc_layer(*common, wa=w1a[l + 1], wb=w1b[l + 1])
        else:
            h = _tc_layer(*common)

    return _tc_pool(h, batch.reshape(n, 1).astype(jnp.int32),
                    r_W1, r_b1.reshape(1, hd), r_W2, r_b2.reshape(1, hd // 2),
                    r_W3, r_b3.reshape(1, 1))


# speedup vs baseline: 4.3414x; 1.1435x over previous
import functools

import jax
import jax.numpy as jnp
from jax import lax
from jax.experimental import pallas as pl
from jax.experimental.pallas import tpu as pltpu
from jax.experimental.pallas import tpu_sc as plsc

_NC = 2
_NS = 16
_LN = 16
_CHUNK = 40
_G = 64

_f32 = jnp.float32


def _sig(v):
    return 1.0 / (1.0 + jnp.exp(-v))


def _dot(a, b):
    return jnp.dot(a, b, preferred_element_type=_f32)


def _tc_encode(x, enc_W, enc_b, wa, wb):
    n, h = x.shape[0], enc_W.shape[1]

    def body(x_ref, w_ref, b_ref, wa_ref, wb_ref, h_ref, a_ref, b2_ref):
        hv = jnp.maximum(_dot(x_ref[...], w_ref[...]) + b_ref[...], 0.0)
        h_ref[...] = hv
        a_ref[...] = _dot(hv, wa_ref[...])
        b2_ref[...] = _dot(hv, wb_ref[...])

    sds = jax.ShapeDtypeStruct((n, h), _f32)
    return pl.pallas_call(body, out_shape=(sds, sds, sds))(x, enc_W, enc_b, wa, wb)


def _tc_edgefeat(edge_attr, edge_W, edge_b, wc, b1):
    e, bf = edge_attr.shape
    h = wc.shape[1]
    blk = 2000

    def body(ea_ref, w_ref, b_ref, wc_ref, b1_ref, o_ref):
        ev = jnp.maximum(_dot(ea_ref[...], w_ref[...]) + b_ref[...], 0.0)
        o_ref[...] = _dot(ev, wc_ref[...]) + b1_ref[...]

    return pl.pallas_call(
        body,
        grid=(e // blk,),
        in_specs=[
            pl.BlockSpec((blk, bf), lambda i: (i, 0)),
            pl.BlockSpec((bf, h), lambda i: (0, 0)),
            pl.BlockSpec((1, h), lambda i: (0, 0)),
            pl.BlockSpec((h, h), lambda i: (0, 0)),
            pl.BlockSpec((1, h), lambda i: (0, 0)),
        ],
        out_specs=pl.BlockSpec((blk, h), lambda i: (i, 0)),
        out_shape=jax.ShapeDtypeStruct((e, h), _f32),
    )(edge_attr, edge_W, edge_b, wc, b1)


def _tc_layer(s2, c2, h, w2, b2, wi, bi, wh, bh, bng, bnb, skw, skb, wa=None, wb=None):
    n, hd = h.shape
    emit_ab = wa is not None

    def body(*refs):
        if emit_ab:
            (s2_ref, c2_ref, h_ref, w2_ref, b2_ref, wi_ref, bi_ref, wh_ref,
             bh_ref, bng_ref, bnb_ref, skw_ref, skb_ref, wa_ref, wb_ref,
             ho_ref, ao_ref, bo_ref) = refs
        else:
            (s2_ref, c2_ref, h_ref, w2_ref, b2_ref, wi_ref, bi_ref, wh_ref,
             bh_ref, bng_ref, bnb_ref, skw_ref, skb_ref, ho_ref) = refs
        s2v = s2_ref[...]
        sv = s2v[0] + s2v[1]
        c2v = c2_ref[...]
        counts = c2v[0, :, 0:1] + c2v[1, :, 0:1]
        hv = h_ref[...]
        aggr = _dot(sv, w2_ref[...]) + counts * b2_ref[...]
        wiv, whv = wi_ref[...], wh_ref[...]
        biv, bhv = bi_ref[...], bh_ref[...]
        r = _sig(_dot(aggr, wiv[:, :hd]) + biv[:, :hd]
                 + _dot(hv, whv[:, :hd]) + bhv[:, :hd])
        z = _sig(_dot(aggr, wiv[:, hd:2 * hd]) + biv[:, hd:2 * hd]
                 + _dot(hv, whv[:, hd:2 * hd]) + bhv[:, hd:2 * hd])
        nn = jnp.tanh(_dot(aggr, wiv[:, 2 * hd:]) + biv[:, 2 * hd:]
                      + r * (_dot(hv, whv[:, 2 * hd:]) + bhv[:, 2 * hd:]))
        hn = (1.0 - z) * nn + z * hv
        mu = jnp.mean(hn, axis=0, keepdims=True)
        var = jnp.mean((hn - mu) ** 2, axis=0, keepdims=True)
        hb = (hn - mu) / jnp.sqrt(var + 1e-5) * bng_ref[...] + bnb_ref[...]
        ho = jnp.maximum(hb, 0.0) + _dot(hv, skw_ref[...]) + skb_ref[...]
        ho_ref[...] = ho
        if emit_ab:
            ao_ref[...] = _dot(ho, wa_ref[...])
            bo_ref[...] = _dot(ho, wb_ref[...])

    sds = jax.ShapeDtypeStruct((n, hd), _f32)
    args = [s2, c2, h, w2, b2, wi, bi, wh, bh, bng, bnb, skw, skb]
    if emit_ab:
        args += [wa, wb]
        return pl.pallas_call(body, out_shape=(sds, sds, sds))(*args)
    return pl.pallas_call(body, out_shape=sds)(*args)


def _tc_pool(h, batch2d, w1, b1, w2, b2, w3, b3):
    n, hd = h.shape

    def body(h_ref, bt_ref, w1_ref, b1_ref, w2_ref, b2_ref, w3_ref, b3_ref, o_ref):
        hv = h_ref[...]
        bt = bt_ref[...]
        gids = lax.broadcasted_iota(jnp.int32, (n, _G), 1)
        onehot = (bt == gids).astype(_f32)
        ssum = lax.dot_general(onehot, hv, (((0,), (0,)), ((), ())),
                               preferred_element_type=_f32)
        cnt = jnp.sum(onehot, axis=0)[:, None]
        hmean = ssum / jnp.maximum(cnt, 1.0)
        rows = []
        for g in range(_G):
            rows.append(jnp.max(jnp.where(bt == g, hv, -3.4e38), axis=0,
                                keepdims=True))
        hmax = jnp.concatenate(rows, axis=0)
        hg = jnp.concatenate([hmean, hmax], axis=1)
        o = jnp.maximum(_dot(hg, w1_ref[...]) + b1_ref[...], 0.0)
        o = jnp.maximum(_dot(o, w2_ref[...]) + b2_ref[...], 0.0)
        o_ref[...] = _sig(_dot(o, w3_ref[...]) + b3_ref[...])

    return pl.pallas_call(
        body, out_shape=jax.ShapeDtypeStruct((_G, 1), _f32),
    )(h, batch2d, w1, b1, w2, b2, w3, b3)


def _sc_counts(dst3d, zc, ones_c):
    nw, ch, ck = dst3d.shape
    n = zc.shape[0] * _NS
    rt = (n // _NS) // 8 * 8
    rem = n - rt * _NS
    mesh = plsc.VectorSubcoreMesh(core_axis_name="c", subcore_axis_name="s")

    @functools.partial(
        pl.kernel, mesh=mesh,
        out_type=jax.ShapeDtypeStruct((_NC, n, _LN), _f32),
        scratch_types=[
            pltpu.VMEM((ch, ck), jnp.int32),
            pltpu.VMEM((ck, _LN), _f32),
            pltpu.VMEM_SHARED((n, _LN), _f32),
        ])
    def k(dst_hbm, zc_hbm, on_hbm, c_out, idxd, onesv, c_sh):
        ci = lax.axis_index("c")
        si = lax.axis_index("s")
        wid = ci * _NS + si
        pltpu.sync_copy(dst_hbm.at[wid], idxd)
        pltpu.sync_copy(on_hbm, onesv)
        pltpu.sync_copy(zc_hbm.at[pl.ds(0, rt)], c_sh.at[pl.ds(si * rt, rt)])

        @pl.when(si == _NS - 1)
        def _():
            pltpu.sync_copy(zc_hbm.at[pl.ds(0, rem)],
                            c_sh.at[pl.ds(_NS * rt, rem)])

        plsc.subcore_barrier()

        @pl.loop(0, ch)
        def _(i):
            pltpu.sync_copy(onesv, c_sh.at[idxd.at[i]], add=True)

        plsc.subcore_barrier()
        pltpu.sync_copy(c_sh.at[pl.ds(si * rt, rt)],
                        c_out.at[ci, pl.ds(si * rt, rt)])

        @pl.when(si == _NS - 1)
        def _():
            pltpu.sync_copy(c_sh.at[pl.ds(_NS * rt, rem)],
                            c_out.at[ci, pl.ds(_NS * rt, rem)])

    return k(dst3d, zc, ones_c)


def _sc_edge_layer(a, b, ec, dst3d, src3d, zs):
    n, hd = a.shape
    nw, ch, ck = dst3d.shape
    ew = ch * ck
    rt = (n // _NS) // 8 * 8
    rem = n - rt * _NS
    mesh = plsc.VectorSubcoreMesh(core_axis_name="c", subcore_axis_name="s")

    @functools.partial(
        pl.kernel, mesh=mesh,
        out_type=jax.ShapeDtypeStruct((_NC, n, hd), _f32),
        scratch_types=[
            pltpu.VMEM((2, ck), jnp.int32),
            pltpu.VMEM((2, ck), jnp.int32),
            pltpu.VMEM((ck, hd), _f32),
            pltpu.VMEM((ck, hd), _f32),
            pltpu.VMEM((ck, hd), _f32),
            pltpu.VMEM((ck, hd), _f32),
            pltpu.VMEM((ck, hd), _f32),
            pltpu.VMEM((ck, hd), _f32),
            pltpu.VMEM_SHARED((n, hd), _f32),
        ] + [pltpu.SemaphoreType.DMA] * 10)
    def k(a_hbm, b_hbm, ec_hbm, dst_hbm, src_hbm, zs_hbm,
          s_out, idxd2, idxs2, av0, av1, bv0, bv1, ecv0, ecv1, s_sh,
          sem_a0, sem_a1, sem_b0, sem_b1, sem_e0, sem_e1,
          sem_d0, sem_d1, sem_s0, sem_s1):
        ci = lax.axis_index("c")
        si = lax.axis_index("s")
        wid = ci * _NS + si
        avs, bvs, ecvs = (av0, av1), (bv0, bv1), (ecv0, ecv1)
        sas, sbs, ses = (sem_a0, sem_a1), (sem_b0, sem_b1), (sem_e0, sem_e1)
        sds, sss = (sem_d0, sem_d1), (sem_s0, sem_s1)

        def issue_idx(j, b):
            pltpu.async_copy(dst_hbm.at[wid, j], idxd2.at[b], sds[b])
            pltpu.async_copy(src_hbm.at[wid, j], idxs2.at[b], sss[b])

        def wait_idx(j, b):
            pltpu.make_async_copy(dst_hbm.at[wid, j], idxd2.at[b],
                                  sds[b]).wait()
            pltpu.make_async_copy(src_hbm.at[wid, j], idxs2.at[b],
                                  sss[b]).wait()

        def issue_gather(j, b):
            pltpu.async_copy(a_hbm.at[idxd2.at[b]], avs[b], sas[b])
            pltpu.async_copy(b_hbm.at[idxs2.at[b]], bvs[b], sbs[b])
            pltpu.async_copy(ec_hbm.at[pl.ds(wid * ew + j * ck, ck)],
                             ecvs[b], ses[b])

        def wait_gather(j, b):
            pltpu.make_async_copy(a_hbm.at[idxd2.at[b]], avs[b],
                                  sas[b]).wait()
            pltpu.make_async_copy(b_hbm.at[idxs2.at[b]], bvs[b],
                                  sbs[b]).wait()
            pltpu.make_async_copy(ec_hbm.at[pl.ds(wid * ew + j * ck, ck)],
                                  ecvs[b], ses[b]).wait()

        pltpu.sync_copy(zs_hbm.at[pl.ds(0, rt)], s_sh.at[pl.ds(si * rt, rt)])

        @pl.when(si == _NS - 1)
        def _():
            pltpu.sync_copy(zs_hbm.at[pl.ds(0, rem)],
                            s_sh.at[pl.ds(_NS * rt, rem)])

        issue_idx(0, 0)
        wait_idx(0, 0)
        issue_gather(0, 0)
        issue_idx(1, 1)
        plsc.subcore_barrier()

        @pl.loop(0, ch, step=2)
        def _(i):
            for b in (0, 1):
                j = i + b
                av, bv, ecv = avs[b], bvs[b], ecvs[b]

                @pl.when(j + 1 < ch)
                def _():
                    wait_idx(j + 1, 1 - b)
                    issue_gather(j + 1, 1 - b)

                wait_gather(j, b)

                @pl.loop(0, ck)
                def _(r):
                    for q in range(hd // _LN):
                        sl = pl.ds(q * _LN, _LN)
                        av[r, sl] = jnp.maximum(
                            av[r, sl] + bv[r, sl] + ecv[r, sl], 0.0)

                pltpu.sync_copy(av, s_sh.at[idxd2.at[b]], add=True)

                @pl.when(j + 2 < ch)
                def _():
                    issue_idx(j + 2, b)

        plsc.subcore_barrier()
        pltpu.sync_copy(s_sh.at[pl.ds(si * rt, rt)],
                        s_out.at[ci, pl.ds(si * rt, rt)])

        @pl.when(si == _NS - 1)
        def _():
            pltpu.sync_copy(s_sh.at[pl.ds(_NS * rt, rem)],
                            s_out.at[ci, pl.ds(_NS * rt, rem)])

    return k(a, b, ec, dst3d, src3d, zs)


def kernel(x, edge_index, edge_attr, batch, enc_W, enc_b, edge_W, edge_b,
           msg_W1, msg_b1, msg_W2, msg_b2, gru_Wi, gru_bi, gru_Wh, gru_bh,
           bn_g, bn_b, skip_W, skip_b, r_W1, r_b1, r_W2, r_b2, r_W3, r_b3):
    n, hd = x.shape[0], enc_W.shape[1]
    e = edge_attr.shape[0]
    nl = msg_W1.shape[0]
    nw = _NC * _NS
    ew = e // nw
    ch = ew // _CHUNK

    src = edge_index[0]
    dst = edge_index[1]
    dst2d = dst.reshape(nw, ch, _CHUNK)
    src2d = src.reshape(nw, ch, _CHUNK)

    w1a = [msg_W1[l, :hd] for l in range(nl)]
    w1b = [msg_W1[l, hd:2 * hd] for l in range(nl)]
    w1c = [msg_W1[l, 2 * hd:] for l in range(nl)]
    b1 = [msg_b1[l].reshape(1, hd) for l in range(nl)]

    zs = jnp.zeros((n // _NS, hd), _f32)
    zc = jnp.zeros((n // _NS, _LN), _f32)
    ones_c = jnp.ones((_CHUNK, _LN), _f32)

    h, a, b = _tc_encode(x, enc_W, enc_b.reshape(1, hd), w1a[0], w1b[0])
    ec = [_tc_edgefeat(edge_attr, edge_W, edge_b.reshape(1, hd), w1c[l], b1[l])
          for l in range(nl)]
    c2 = _sc_counts(dst2d, zc, ones_c)

    for l in range(nl):
        s2 = _sc_edge_layer(a, b, ec[l], dst2d, src2d, zs)
        common = (s2, c2, h, msg_W2[l], msg_b2[l].reshape(1, hd),
                  gru_Wi[l], gru_bi[l].reshape(1, 3 * hd),
                  gru_Wh[l], gru_bh[l].reshape(1, 3 * hd),
                  bn_g[l].reshape(1, hd), bn_b[l].reshape(1, hd),
                  skip_W[l], skip_b[l].reshape(1, hd))
        if l < nl - 1:
            h, a, b = _tc_layer(*common, wa=w1a[l + 1], wb=w1b[l + 1])
        else:
            h = _tc_layer(*common)

    return _tc_pool(h, batch.reshape(n, 1).astype(jnp.int32),
                    r_W1, r_b1.reshape(1, hd), r_W2, r_b2.reshape(1, hd // 2),
                    r_W3, r_b3.reshape(1, 1))


# speedup vs baseline: 4.5908x; 1.0574x over previous
import functools

import jax
import jax.numpy as jnp
from jax import lax
from jax.experimental import pallas as pl
from jax.experimental.pallas import tpu as pltpu
from jax.experimental.pallas import tpu_sc as plsc

_NC = 2
_NS = 16
_LN = 16
_CHUNK = 40
_G = 64

_f32 = jnp.float32


def _sig(v):
    return 1.0 / (1.0 + jnp.exp(-v))


def _dot(a, b):
    return jnp.dot(a, b, preferred_element_type=_f32)


def _tc_encode(x, enc_W, enc_b, wa, wb):
    n, h = x.shape[0], enc_W.shape[1]

    def body(x_ref, w_ref, b_ref, wa_ref, wb_ref, h_ref, a_ref, b2_ref):
        hv = jnp.maximum(_dot(x_ref[...], w_ref[...]) + b_ref[...], 0.0)
        h_ref[...] = hv
        a_ref[...] = _dot(hv, wa_ref[...])
        b2_ref[...] = _dot(hv, wb_ref[...])

    sds = jax.ShapeDtypeStruct((n, h), _f32)
    return pl.pallas_call(body, out_shape=(sds, sds, sds))(x, enc_W, enc_b, wa, wb)


def _tc_edgefeat(edge_attr, edge_W, edge_b, wc, b1):
    e, bf = edge_attr.shape
    h = wc.shape[1]
    blk = 2000

    def body(ea_ref, w_ref, b_ref, wc_ref, b1_ref, o_ref):
        ev = jnp.maximum(_dot(ea_ref[...], w_ref[...]) + b_ref[...], 0.0)
        o_ref[...] = _dot(ev, wc_ref[...]) + b1_ref[...]

    return pl.pallas_call(
        body,
        grid=(e // blk,),
        in_specs=[
            pl.BlockSpec((blk, bf), lambda i: (i, 0)),
            pl.BlockSpec((bf, h), lambda i: (0, 0)),
            pl.BlockSpec((1, h), lambda i: (0, 0)),
            pl.BlockSpec((h, h), lambda i: (0, 0)),
            pl.BlockSpec((1, h), lambda i: (0, 0)),
        ],
        out_specs=pl.BlockSpec((blk, h), lambda i: (i, 0)),
        out_shape=jax.ShapeDtypeStruct((e, h), _f32),
    )(edge_attr, edge_W, edge_b, wc, b1)


def _tc_layer(s2, c2, h, w2, b2, wi, bi, wh, bh, bng, bnb, skw, skb, wa=None, wb=None):
    n, hd = h.shape
    emit_ab = wa is not None

    def body(*refs):
        if emit_ab:
            (s2_ref, c2_ref, h_ref, w2_ref, b2_ref, wi_ref, bi_ref, wh_ref,
             bh_ref, bng_ref, bnb_ref, skw_ref, skb_ref, wa_ref, wb_ref,
             ho_ref, ao_ref, bo_ref) = refs
        else:
            (s2_ref, c2_ref, h_ref, w2_ref, b2_ref, wi_ref, bi_ref, wh_ref,
             bh_ref, bng_ref, bnb_ref, skw_ref, skb_ref, ho_ref) = refs
        s2v = s2_ref[...]
        sv = s2v[0] + s2v[1]
        c2v = c2_ref[...]
        counts = c2v[0, :, 0:1] + c2v[1, :, 0:1]
        hv = h_ref[...]
        aggr = _dot(sv, w2_ref[...]) + counts * b2_ref[...]
        wiv, whv = wi_ref[...], wh_ref[...]
        biv, bhv = bi_ref[...], bh_ref[...]
        r = _sig(_dot(aggr, wiv[:, :hd]) + biv[:, :hd]
                 + _dot(hv, whv[:, :hd]) + bhv[:, :hd])
        z = _sig(_dot(aggr, wiv[:, hd:2 * hd]) + biv[:, hd:2 * hd]
                 + _dot(hv, whv[:, hd:2 * hd]) + bhv[:, hd:2 * hd])
        nn = jnp.tanh(_dot(aggr, wiv[:, 2 * hd:]) + biv[:, 2 * hd:]
                      + r * (_dot(hv, whv[:, 2 * hd:]) + bhv[:, 2 * hd:]))
        hn = (1.0 - z) * nn + z * hv
        mu = jnp.mean(hn, axis=0, keepdims=True)
        var = jnp.mean((hn - mu) ** 2, axis=0, keepdims=True)
        hb = (hn - mu) / jnp.sqrt(var + 1e-5) * bng_ref[...] + bnb_ref[...]
        ho = jnp.maximum(hb, 0.0) + _dot(hv, skw_ref[...]) + skb_ref[...]
        ho_ref[...] = ho
        if emit_ab:
            ao_ref[...] = _dot(ho, wa_ref[...])
            bo_ref[...] = _dot(ho, wb_ref[...])

    sds = jax.ShapeDtypeStruct((n, hd), _f32)
    args = [s2, c2, h, w2, b2, wi, bi, wh, bh, bng, bnb, skw, skb]
    if emit_ab:
        args += [wa, wb]
        return pl.pallas_call(body, out_shape=(sds, sds, sds))(*args)
    return pl.pallas_call(body, out_shape=sds)(*args)


def _tc_pool(h, batch2d, w1, b1, w2, b2, w3, b3):
    n, hd = h.shape

    def body(h_ref, bt_ref, w1_ref, b1_ref, w2_ref, b2_ref, w3_ref, b3_ref, o_ref):
        hv = h_ref[...]
        bt = bt_ref[...]
        gids = lax.broadcasted_iota(jnp.int32, (n, _G), 1)
        onehot = (bt == gids).astype(_f32)
        ssum = lax.dot_general(onehot, hv, (((0,), (0,)), ((), ())),
                               preferred_element_type=_f32)
        cnt = jnp.sum(onehot, axis=0)[:, None]
        hmean = ssum / jnp.maximum(cnt, 1.0)
        rows = []
        for g in range(_G):
            rows.append(jnp.max(jnp.where(bt == g, hv, -3.4e38), axis=0,
                                keepdims=True))
        hmax = jnp.concatenate(rows, axis=0)
        hg = jnp.concatenate([hmean, hmax], axis=1)
        o = jnp.maximum(_dot(hg, w1_ref[...]) + b1_ref[...], 0.0)
        o = jnp.maximum(_dot(o, w2_ref[...]) + b2_ref[...], 0.0)
        o_ref[...] = _sig(_dot(o, w3_ref[...]) + b3_ref[...])

    return pl.pallas_call(
        body, out_shape=jax.ShapeDtypeStruct((_G, 1), _f32),
    )(h, batch2d, w1, b1, w2, b2, w3, b3)


def _sc_counts(dst3d, zc, ones_c):
    nw, ch, ck = dst3d.shape
    n = zc.shape[0] * _NS
    rt = (n // _NS) // 8 * 8
    rem = n - rt * _NS
    mesh = plsc.VectorSubcoreMesh(core_axis_name="c", subcore_axis_name="s")

    @functools.partial(
        pl.kernel, mesh=mesh,
        out_type=jax.ShapeDtypeStruct((_NC, n, _LN), _f32),
        scratch_types=[
            pltpu.VMEM((ch, ck), jnp.int32),
            pltpu.VMEM((ck, _LN), _f32),
            pltpu.VMEM_SHARED((n, _LN), _f32),
        ])
    def k(dst_hbm, zc_hbm, on_hbm, c_out, idxd, onesv, c_sh):
        ci = lax.axis_index("c")
        si = lax.axis_index("s")
        wid = ci * _NS + si
        pltpu.sync_copy(dst_hbm.at[wid], idxd)
        pltpu.sync_copy(on_hbm, onesv)
        pltpu.sync_copy(zc_hbm.at[pl.ds(0, rt)], c_sh.at[pl.ds(si * rt, rt)])

        @pl.when(si == _NS - 1)
        def _():
            pltpu.sync_copy(zc_hbm.at[pl.ds(0, rem)],
                            c_sh.at[pl.ds(_NS * rt, rem)])

        plsc.subcore_barrier()

        @pl.loop(0, ch)
        def _(i):
            pltpu.sync_copy(onesv, c_sh.at[idxd.at[i]], add=True)

        plsc.subcore_barrier()
        pltpu.sync_copy(c_sh.at[pl.ds(si * rt, rt)],
                        c_out.at[ci, pl.ds(si * rt, rt)])

        @pl.when(si == _NS - 1)
        def _():
            pltpu.sync_copy(c_sh.at[pl.ds(_NS * rt, rem)],
                            c_out.at[ci, pl.ds(_NS * rt, rem)])

    return k(dst3d, zc, ones_c)


def _sc_edge_layer(a, b, ec, dst3d, src3d, zs):
    n, hd = a.shape
    nw, ch, ck = dst3d.shape
    ew = ch * ck
    rt = (n // _NS) // 8 * 8
    rem = n - rt * _NS
    mesh = plsc.VectorSubcoreMesh(core_axis_name="c", subcore_axis_name="s")

    @functools.partial(
        pl.kernel, mesh=mesh,
        out_type=jax.ShapeDtypeStruct((_NC, n, hd), _f32),
        scratch_types=[
            pltpu.VMEM((4, ck), jnp.int32),
            pltpu.VMEM((4, ck), jnp.int32),
            pltpu.VMEM((ck, hd), _f32),
            pltpu.VMEM((ck, hd), _f32),
            pltpu.VMEM((ck, hd), _f32),
            pltpu.VMEM((ck, hd), _f32),
            pltpu.VMEM((ck, hd), _f32),
            pltpu.VMEM((ck, hd), _f32),
            pltpu.VMEM_SHARED((n, hd), _f32),
        ] + [pltpu.SemaphoreType.DMA] * 9)
    def k(a_hbm, b_hbm, ec_hbm, dst_hbm, src_hbm, zs_hbm,
          s_out, idxd4, idxs4, av0, av1, bv0, bv1, ecv0, ecv1, s_sh,
          sem_a0, sem_a1, sem_b0, sem_b1, sem_e0, sem_e1,
          sem_i, sem_w0, sem_w1):
        ci = lax.axis_index("c")
        si = lax.axis_index("s")
        wid = ci * _NS + si
        avs, bvs, ecvs = (av0, av1), (bv0, bv1), (ecv0, ecv1)
        sas, sbs, ses = (sem_a0, sem_a1), (sem_b0, sem_b1), (sem_e0, sem_e1)
        sws = (sem_w0, sem_w1)

        def issue_idx(j):
            s = lax.rem(j, 4)
            pltpu.async_copy(dst_hbm.at[wid, j], idxd4.at[s], sem_i)
            pltpu.async_copy(src_hbm.at[wid, j], idxs4.at[s], sem_i)

        def wait_idx(j):
            s = lax.rem(j, 4)
            pltpu.make_async_copy(dst_hbm.at[wid, j], idxd4.at[s],
                                  sem_i).wait()
            pltpu.make_async_copy(src_hbm.at[wid, j], idxs4.at[s],
                                  sem_i).wait()

        def issue_gather(j, b):
            s = lax.rem(j, 4)
            pltpu.async_copy(a_hbm.at[idxd4.at[s]], avs[b], sas[b])
            pltpu.async_copy(b_hbm.at[idxs4.at[s]], bvs[b], sbs[b])
            pltpu.async_copy(ec_hbm.at[pl.ds(wid * ew + j * ck, ck)],
                             ecvs[b], ses[b])

        def wait_gather(j, b):
            s = lax.rem(j, 4)
            pltpu.make_async_copy(a_hbm.at[idxd4.at[s]], avs[b],
                                  sas[b]).wait()
            pltpu.make_async_copy(b_hbm.at[idxs4.at[s]], bvs[b],
                                  sbs[b]).wait()
            pltpu.make_async_copy(ec_hbm.at[pl.ds(wid * ew + j * ck, ck)],
                                  ecvs[b], ses[b]).wait()

        def wait_scatter(j, b):
            s = lax.rem(j, 4)
            pltpu.make_async_copy(avs[b], s_sh.at[idxd4.at[s]],
                                  sws[b]).wait()

        pltpu.sync_copy(zs_hbm.at[pl.ds(0, rt)], s_sh.at[pl.ds(si * rt, rt)])

        @pl.when(si == _NS - 1)
        def _():
            pltpu.sync_copy(zs_hbm.at[pl.ds(0, rem)],
                            s_sh.at[pl.ds(_NS * rt, rem)])

        issue_idx(0)
        wait_idx(0)
        issue_gather(0, 0)
        issue_idx(1)
        plsc.subcore_barrier()

        @pl.loop(0, ch, step=2)
        def _(i):
            for b in (0, 1):
                j = i + b
                av, bv, ecv = avs[b], bvs[b], ecvs[b]

                @pl.when(j + 1 < ch)
                def _():
                    wait_idx(j + 1)

                    @pl.when(j >= 1)
                    def _():
                        wait_scatter(j - 1, 1 - b)

                    issue_gather(j + 1, 1 - b)

                wait_gather(j, b)

                @pl.loop(0, ck)
                def _(r):
                    for q in range(hd // _LN):
                        sl = pl.ds(q * _LN, _LN)
                        av[r, sl] = jnp.maximum(
                            av[r, sl] + bv[r, sl] + ecv[r, sl], 0.0)

                pltpu.async_copy(av, s_sh.at[idxd4.at[lax.rem(j, 4)]],
                                 sws[b], add=True)

                @pl.when(j + 2 < ch)
                def _():
                    issue_idx(j + 2)

        wait_scatter(ch - 2, 0)
        wait_scatter(ch - 1, 1)
        plsc.subcore_barrier()
        pltpu.sync_copy(s_sh.at[pl.ds(si * rt, rt)],
                        s_out.at[ci, pl.ds(si * rt, rt)])

        @pl.when(si == _NS - 1)
        def _():
            pltpu.sync_copy(s_sh.at[pl.ds(_NS * rt, rem)],
                            s_out.at[ci, pl.ds(_NS * rt, rem)])

    return k(a, b, ec, dst3d, src3d, zs)


def kernel(x, edge_index, edge_attr, batch, enc_W, enc_b, edge_W, edge_b,
           msg_W1, msg_b1, msg_W2, msg_b2, gru_Wi, gru_bi, gru_Wh, gru_bh,
           bn_g, bn_b, skip_W, skip_b, r_W1, r_b1, r_W2, r_b2, r_W3, r_b3):
    n, hd = x.shape[0], enc_W.shape[1]
    e = edge_attr.shape[0]
    nl = msg_W1.shape[0]
    nw = _NC * _NS
    ew = e // nw
    ch = ew // _CHUNK

    src = edge_index[0]
    dst = edge_index[1]
    dst2d = dst.reshape(nw, ch, _CHUNK)
    src2d = src.reshape(nw, ch, _CHUNK)

    w1a = [msg_W1[l, :hd] for l in range(nl)]
    w1b = [msg_W1[l, hd:2 * hd] for l in range(nl)]
    w1c = [msg_W1[l, 2 * hd:] for l in range(nl)]
    b1 = [msg_b1[l].reshape(1, hd) for l in range(nl)]

    zs = jnp.zeros((n // _NS, hd), _f32)
    zc = jnp.zeros((n // _NS, _LN), _f32)
    ones_c = jnp.ones((_CHUNK, _LN), _f32)

    h, a, b = _tc_encode(x, enc_W, enc_b.reshape(1, hd), w1a[0], w1b[0])
    ec = [_tc_edgefeat(edge_attr, edge_W, edge_b.reshape(1, hd), w1c[l], b1[l])
          for l in range(nl)]
    c2 = _sc_counts(dst2d, zc, ones_c)

    for l in range(nl):
        s2 = _sc_edge_layer(a, b, ec[l], dst2d, src2d, zs)
        common = (s2, c2, h, msg_W2[l], msg_b2[l].reshape(1, hd),
                  gru_Wi[l], gru_bi[l].reshape(1, 3 * hd),
                  gru_Wh[l], gru_bh[l].reshape(1, 3 * hd),
                  bn_g[l].reshape(1, hd), bn_b[l].reshape(1, hd),
                  skip_W[l], skip_b[l].reshape(1, hd))
        if l < nl - 1:
            h, a, b = _tc_layer(*common, wa=w1a[l + 1], wb=w1b[l + 1])
        else:
            h = _tc_layer(*common)

    return _tc_pool(h, batch.reshape(n, 1).astype(jnp.int32),
                    r_W1, r_b1.reshape(1, hd), r_W2, r_b2.reshape(1, hd // 2),
                    r_W3, r_b3.reshape(1, 1))


# speedup vs baseline: 4.6641x; 1.0160x over previous
import functools

import jax
import jax.numpy as jnp
from jax import lax
from jax.experimental import pallas as pl
from jax.experimental.pallas import tpu as pltpu
from jax.experimental.pallas import tpu_sc as plsc

_NC = 2
_NS = 16
_LN = 16
_CHUNK = 40
_G = 64

_f32 = jnp.float32


def _sig(v):
    return 1.0 / (1.0 + jnp.exp(-v))


def _dot(a, b):
    return jnp.dot(a, b, preferred_element_type=_f32)


def _tc_encode(x, enc_W, enc_b, wa, wb):
    n, h = x.shape[0], enc_W.shape[1]

    def body(x_ref, w_ref, b_ref, wa_ref, wb_ref, h_ref, ab_ref):
        hv = jnp.maximum(_dot(x_ref[...], w_ref[...]) + b_ref[...], 0.0)
        h_ref[...] = hv
        ab_ref[0:n] = _dot(hv, wa_ref[...])
        ab_ref[n:2 * n] = _dot(hv, wb_ref[...])

    return pl.pallas_call(
        body, out_shape=(jax.ShapeDtypeStruct((n, h), _f32),
                         jax.ShapeDtypeStruct((2 * n, h), _f32)),
    )(x, enc_W, enc_b, wa, wb)


def _tc_edgefeat(edge_attr, edge_W, edge_b, wc, b1):
    e, bf = edge_attr.shape
    h = wc.shape[1]
    blk = 2000

    def body(ea_ref, w_ref, b_ref, wc_ref, b1_ref, o_ref):
        ev = jnp.maximum(_dot(ea_ref[...], w_ref[...]) + b_ref[...], 0.0)
        o_ref[...] = _dot(ev, wc_ref[...]) + b1_ref[...]

    return pl.pallas_call(
        body,
        grid=(e // blk,),
        in_specs=[
            pl.BlockSpec((blk, bf), lambda i: (i, 0)),
            pl.BlockSpec((bf, h), lambda i: (0, 0)),
            pl.BlockSpec((1, h), lambda i: (0, 0)),
            pl.BlockSpec((h, h), lambda i: (0, 0)),
            pl.BlockSpec((1, h), lambda i: (0, 0)),
        ],
        out_specs=pl.BlockSpec((blk, h), lambda i: (i, 0)),
        out_shape=jax.ShapeDtypeStruct((e, h), _f32),
    )(edge_attr, edge_W, edge_b, wc, b1)


def _tc_layer(s2, c2, h, w2, b2, wi, bi, wh, bh, bng, bnb, skw, skb, wa=None, wb=None):
    n, hd = h.shape
    emit_ab = wa is not None

    def body(*refs):
        if emit_ab:
            (s2_ref, c2_ref, h_ref, w2_ref, b2_ref, wi_ref, bi_ref, wh_ref,
             bh_ref, bng_ref, bnb_ref, skw_ref, skb_ref, wa_ref, wb_ref,
             ho_ref, ab_ref) = refs
        else:
            (s2_ref, c2_ref, h_ref, w2_ref, b2_ref, wi_ref, bi_ref, wh_ref,
             bh_ref, bng_ref, bnb_ref, skw_ref, skb_ref, ho_ref) = refs
        s2v = s2_ref[...]
        sv = s2v[0] + s2v[1]
        c2v = c2_ref[...]
        counts = c2v[0, :, 0:1] + c2v[1, :, 0:1]
        hv = h_ref[...]
        aggr = _dot(sv, w2_ref[...]) + counts * b2_ref[...]
        wiv, whv = wi_ref[...], wh_ref[...]
        biv, bhv = bi_ref[...], bh_ref[...]
        r = _sig(_dot(aggr, wiv[:, :hd]) + biv[:, :hd]
                 + _dot(hv, whv[:, :hd]) + bhv[:, :hd])
        z = _sig(_dot(aggr, wiv[:, hd:2 * hd]) + biv[:, hd:2 * hd]
                 + _dot(hv, whv[:, hd:2 * hd]) + bhv[:, hd:2 * hd])
        nn = jnp.tanh(_dot(aggr, wiv[:, 2 * hd:]) + biv[:, 2 * hd:]
                      + r * (_dot(hv, whv[:, 2 * hd:]) + bhv[:, 2 * hd:]))
        hn = (1.0 - z) * nn + z * hv
        mu = jnp.mean(hn, axis=0, keepdims=True)
        var = jnp.mean((hn - mu) ** 2, axis=0, keepdims=True)
        hb = (hn - mu) / jnp.sqrt(var + 1e-5) * bng_ref[...] + bnb_ref[...]
        ho = jnp.maximum(hb, 0.0) + _dot(hv, skw_ref[...]) + skb_ref[...]
        ho_ref[...] = ho
        if emit_ab:
            ab_ref[0:n] = _dot(ho, wa_ref[...])
            ab_ref[n:2 * n] = _dot(ho, wb_ref[...])

    sds = jax.ShapeDtypeStruct((n, hd), _f32)
    args = [s2, c2, h, w2, b2, wi, bi, wh, bh, bng, bnb, skw, skb]
    if emit_ab:
        args += [wa, wb]
        return pl.pallas_call(
            body, out_shape=(sds, jax.ShapeDtypeStruct((2 * n, hd), _f32)),
        )(*args)
    return pl.pallas_call(body, out_shape=sds)(*args)


def _tc_pool(h, batch2d, w1, b1, w2, b2, w3, b3):
    n, hd = h.shape

    def body(h_ref, bt_ref, w1_ref, b1_ref, w2_ref, b2_ref, w3_ref, b3_ref, o_ref):
        hv = h_ref[...]
        bt = bt_ref[...]
        gids = lax.broadcasted_iota(jnp.int32, (n, _G), 1)
        onehot = (bt == gids).astype(_f32)
        ssum = lax.dot_general(onehot, hv, (((0,), (0,)), ((), ())),
                               preferred_element_type=_f32)
        cnt = jnp.sum(onehot, axis=0)[:, None]
        hmean = ssum / jnp.maximum(cnt, 1.0)
        rows = []
        for g in range(_G):
            rows.append(jnp.max(jnp.where(bt == g, hv, -3.4e38), axis=0,
                                keepdims=True))
        hmax = jnp.concatenate(rows, axis=0)
        hg = jnp.concatenate([hmean, hmax], axis=1)
        o = jnp.maximum(_dot(hg, w1_ref[...]) + b1_ref[...], 0.0)
        o = jnp.maximum(_dot(o, w2_ref[...]) + b2_ref[...], 0.0)
        o_ref[...] = _sig(_dot(o, w3_ref[...]) + b3_ref[...])

    return pl.pallas_call(
        body, out_shape=jax.ShapeDtypeStruct((_G, 1), _f32),
    )(h, batch2d, w1, b1, w2, b2, w3, b3)


def _sc_counts(dst3d, zc, ones_c):
    nw, ch, ck = dst3d.shape
    n = zc.shape[0] * _NS
    rt = (n // _NS) // 8 * 8
    rem = n - rt * _NS
    mesh = plsc.VectorSubcoreMesh(core_axis_name="c", subcore_axis_name="s")

    @functools.partial(
        pl.kernel, mesh=mesh,
        out_type=jax.ShapeDtypeStruct((_NC, n, _LN), _f32),
        scratch_types=[
            pltpu.VMEM((ch, ck), jnp.int32),
            pltpu.VMEM((ck, _LN), _f32),
            pltpu.VMEM_SHARED((n, _LN), _f32),
        ])
    def k(dst_hbm, zc_hbm, on_hbm, c_out, idxd, onesv, c_sh):
        ci = lax.axis_index("c")
        si = lax.axis_index("s")
        wid = ci * _NS + si
        pltpu.sync_copy(dst_hbm.at[wid], idxd)
        pltpu.sync_copy(on_hbm, onesv)
        pltpu.sync_copy(zc_hbm.at[pl.ds(0, rt)], c_sh.at[pl.ds(si * rt, rt)])

        @pl.when(si == _NS - 1)
        def _():
            pltpu.sync_copy(zc_hbm.at[pl.ds(0, rem)],
                            c_sh.at[pl.ds(_NS * rt, rem)])

        plsc.subcore_barrier()

        @pl.loop(0, ch)
        def _(i):
            pltpu.sync_copy(onesv, c_sh.at[idxd.at[i]], add=True)

        plsc.subcore_barrier()
        pltpu.sync_copy(c_sh.at[pl.ds(si * rt, rt)],
                        c_out.at[ci, pl.ds(si * rt, rt)])

        @pl.when(si == _NS - 1)
        def _():
            pltpu.sync_copy(c_sh.at[pl.ds(_NS * rt, rem)],
                            c_out.at[ci, pl.ds(_NS * rt, rem)])

    return k(dst3d, zc, ones_c)


def _sc_edge_layer(ab, ec, comb3d, zs):
    n2, hd = ab.shape
    n = n2 // 2
    nw, ch, ck2 = comb3d.shape
    ck = ck2 // 2
    ew = ch * ck
    rt = (n // _NS) // 8 * 8
    rem = n - rt * _NS
    mesh = plsc.VectorSubcoreMesh(core_axis_name="c", subcore_axis_name="s")

    @functools.partial(
        pl.kernel, mesh=mesh,
        out_type=jax.ShapeDtypeStruct((_NC, n, hd), _f32),
        scratch_types=[
            pltpu.VMEM((4, ck2), jnp.int32),
            pltpu.VMEM((ck2, hd), _f32),
            pltpu.VMEM((ck2, hd), _f32),
            pltpu.VMEM((ck, hd), _f32),
            pltpu.VMEM((ck, hd), _f32),
            pltpu.VMEM_SHARED((n, hd), _f32),
        ] + [pltpu.SemaphoreType.DMA] * 7)
    def k(ab_hbm, ec_hbm, comb_hbm, zs_hbm,
          s_out, idxc4, abv0, abv1, ecv0, ecv1, s_sh,
          sem_ab0, sem_ab1, sem_e0, sem_e1, sem_i, sem_w0, sem_w1):
        ci = lax.axis_index("c")
        si = lax.axis_index("s")
        wid = ci * _NS + si
        abvs, ecvs = (abv0, abv1), (ecv0, ecv1)
        sabs, ses = (sem_ab0, sem_ab1), (sem_e0, sem_e1)
        sws = (sem_w0, sem_w1)

        def issue_idx(j):
            s = lax.rem(j, 4)
            pltpu.async_copy(comb_hbm.at[wid, j], idxc4.at[s], sem_i)

        def wait_idx(j):
            s = lax.rem(j, 4)
            pltpu.make_async_copy(comb_hbm.at[wid, j], idxc4.at[s],
                                  sem_i).wait()

        def issue_gather(j, b):
            s = lax.rem(j, 4)
            pltpu.async_copy(ab_hbm.at[idxc4.at[s]], abvs[b], sabs[b])
            pltpu.async_copy(ec_hbm.at[pl.ds(wid * ew + j * ck, ck)],
                             ecvs[b], ses[b])

        def wait_gather(j, b):
            s = lax.rem(j, 4)
            pltpu.make_async_copy(ab_hbm.at[idxc4.at[s]], abvs[b],
                                  sabs[b]).wait()
            pltpu.make_async_copy(ec_hbm.at[pl.ds(wid * ew + j * ck, ck)],
                                  ecvs[b], ses[b]).wait()

        def wait_scatter(j, b):
            s = lax.rem(j, 4)
            pltpu.make_async_copy(ecvs[b],
                                  s_sh.at[idxc4.at[s, pl.ds(0, ck)]],
                                  sws[b]).wait()

        pltpu.sync_copy(zs_hbm.at[pl.ds(0, rt)], s_sh.at[pl.ds(si * rt, rt)])

        @pl.when(si == _NS - 1)
        def _():
            pltpu.sync_copy(zs_hbm.at[pl.ds(0, rem)],
                            s_sh.at[pl.ds(_NS * rt, rem)])

        issue_idx(0)
        wait_idx(0)
        issue_gather(0, 0)
        issue_idx(1)
        plsc.subcore_barrier()

        @pl.loop(0, ch, step=2)
        def _(i):
            for b in (0, 1):
                j = i + b
                abv, ecv = abvs[b], ecvs[b]

                @pl.when(j + 1 < ch)
                def _():
                    wait_idx(j + 1)

                    @pl.when(j >= 1)
                    def _():
                        wait_scatter(j - 1, 1 - b)

                    issue_gather(j + 1, 1 - b)

                wait_gather(j, b)

                @pl.loop(0, ck, step=2)
                def _(r):
                    for rr in (0, 1):
                        for q in range(hd // _LN):
                            sl = pl.ds(q * _LN, _LN)
                            ecv[r + rr, sl] = jnp.maximum(
                                abv[r + rr, sl] + abv[ck + r + rr, sl]
                                + ecv[r + rr, sl], 0.0)

                pltpu.async_copy(
                    ecv, s_sh.at[idxc4.at[lax.rem(j, 4), pl.ds(0, ck)]],
                    sws[b], add=True)

                @pl.when(j + 2 < ch)
                def _():
                    issue_idx(j + 2)

        wait_scatter(ch - 2, 0)
        wait_scatter(ch - 1, 1)
        plsc.subcore_barrier()
        pltpu.sync_copy(s_sh.at[pl.ds(si * rt, rt)],
                        s_out.at[ci, pl.ds(si * rt, rt)])

        @pl.when(si == _NS - 1)
        def _():
            pltpu.sync_copy(s_sh.at[pl.ds(_NS * rt, rem)],
                            s_out.at[ci, pl.ds(_NS * rt, rem)])

    return k(ab, ec, comb3d, zs)


def kernel(x, edge_index, edge_attr, batch, enc_W, enc_b, edge_W, edge_b,
           msg_W1, msg_b1, msg_W2, msg_b2, gru_Wi, gru_bi, gru_Wh, gru_bh,
           bn_g, bn_b, skip_W, skip_b, r_W1, r_b1, r_W2, r_b2, r_W3, r_b3):
    n, hd = x.shape[0], enc_W.shape[1]
    e = edge_attr.shape[0]
    nl = msg_W1.shape[0]
    nw = _NC * _NS
    ew = e // nw
    ch = ew // _CHUNK

    src = edge_index[0]
    dst = edge_index[1]
    dst2d = dst.reshape(nw, ch, _CHUNK)
    src2d = src.reshape(nw, ch, _CHUNK)
    comb3d = jnp.concatenate([dst2d, src2d + n], axis=2)

    w1a = [msg_W1[l, :hd] for l in range(nl)]
    w1b = [msg_W1[l, hd:2 * hd] for l in range(nl)]
    w1c = [msg_W1[l, 2 * hd:] for l in range(nl)]
    b1 = [msg_b1[l].reshape(1, hd) for l in range(nl)]

    zs = jnp.zeros((n // _NS, hd), _f32)
    zc = jnp.zeros((n // _NS, _LN), _f32)
    ones_c = jnp.ones((_CHUNK, _LN), _f32)

    h, ab = _tc_encode(x, enc_W, enc_b.reshape(1, hd), w1a[0], w1b[0])
    ec = [_tc_edgefeat(edge_attr, edge_W, edge_b.reshape(1, hd), w1c[l], b1[l])
          for l in range(nl)]
    c2 = _sc_counts(dst2d, zc, ones_c)

    for l in range(nl):
        s2 = _sc_edge_layer(ab, ec[l], comb3d, zs)
        common = (s2, c2, h, msg_W2[l], msg_b2[l].reshape(1, hd),
                  gru_Wi[l], gru_bi[l].reshape(1, 3 * hd),
                  gru_Wh[l], gru_bh[l].reshape(1, 3 * hd),
                  bn_g[l].reshape(1, hd), bn_b[l].reshape(1, hd),
                  skip_W[l], skip_b[l].reshape(1, hd))
        if l < nl - 1:
            h, ab = _tc_layer(*common, wa=w1a[l + 1], wb=w1b[l + 1])
        else:
            h = _tc_layer(*common)

    return _tc_pool(h, batch.reshape(n, 1).astype(jnp.int32),
                    r_W1, r_b1.reshape(1, hd), r_W2, r_b2.reshape(1, hd // 2),
                    r_W3, r_b3.reshape(1, 1))
